# baseline (speedup 1.0000x reference)
import functools

import jax
import jax.numpy as jnp
from jax import lax
from jax.experimental import pallas as pl
from jax.experimental.pallas import tpu as pltpu
from jax.experimental.pallas import tpu_sc as plsc

N = 10000
E = 160000
D = 256
H = 128
NP = 10240
NC = 2
NS = 16
ECH = 128
NCH = E // ECH
CPS = NCH // NS
NXTRA = NCH - CPS * NS
RPS = NP // NS
BM = 1024
MB = NP // BM
CPC = NCH // NC
DCPS = CPC // NS


def _sc_mesh():
    return plsc.VectorSubcoreMesh(
        core_axis_name="c", subcore_axis_name="s",
        num_cores=NC, num_subcores=NS)


def _deg(dst):
    @functools.partial(
        pl.kernel,
        out_type=jax.ShapeDtypeStruct((NC * NP,), jnp.float32),
        mesh=_sc_mesh(),
        scratch_types=[
            pltpu.VMEM((ECH,), jnp.int32),
            pltpu.VMEM((ECH,), jnp.int32),
            pltpu.VMEM((ECH,), jnp.float32),
            pltpu.VMEM((RPS,), jnp.float32),
            pltpu.VMEM_SHARED((NP,), jnp.float32),
            pltpu.SemaphoreType.DMA,
            pltpu.SemaphoreType.DMA,
        ],
    )
    def k(dst_hbm, deg_hbm, didx0, didx1, ones_v, val_v, deg_sh, sem0, sem1):
        c = lax.axis_index("c")
        s = lax.axis_index("s")

        @pl.loop(0, RPS // 16)
        def _(i):
            val_v[pl.ds(i * 16, 16)] = jnp.zeros((16,), jnp.float32)

        pltpu.sync_copy(val_v, deg_sh.at[pl.ds(s * RPS, RPS)])

        @pl.loop(0, ECH // 16)
        def _(i):
            ones_v[pl.ds(i * 16, 16)] = jnp.ones((16,), jnp.float32)

        plsc.subcore_barrier()

        ebase = (c * CPC + s * DCPS) * ECH
        pltpu.sync_copy(dst_hbm.at[pl.ds(ebase, ECH)], didx0)

        @pl.loop(0, DCPS, step=2)
        def _(i):
            pltpu.async_copy(ones_v, deg_sh.at[didx0], sem0, add=True)

            @pl.when(i + 1 < DCPS)
            def _():
                pltpu.sync_copy(
                    dst_hbm.at[pl.ds(ebase + (i + 1) * ECH, ECH)], didx1)

            pltpu.make_async_copy(ones_v, deg_sh.at[didx0], sem0).wait()

            @pl.when(i + 1 < DCPS)
            def _():
                pltpu.async_copy(ones_v, deg_sh.at[didx1], sem1, add=True)

                @pl.when(i + 2 < DCPS)
                def _():
                    pltpu.sync_copy(
                        dst_hbm.at[pl.ds(ebase + (i + 2) * ECH, ECH)], didx0)

                pltpu.make_async_copy(ones_v, deg_sh.at[didx1], sem1).wait()

        @pl.when(s == 0)
        def _():
            pltpu.sync_copy(
                dst_hbm.at[pl.ds((c * CPC + NS * DCPS) * ECH, ECH)], didx0)
            pltpu.sync_copy(ones_v, deg_sh.at[didx0], add=True)

        plsc.subcore_barrier()
        pltpu.sync_copy(deg_sh.at[pl.ds(s * RPS, RPS)],
                        deg_hbm.at[pl.ds(c * NP + s * RPS, RPS)])

    return k(dst)


PCH = 80
PCPS = E // PCH // NS


def _prop_body(g_hbm, src2_hbm, dst_hbm, zer_hbm, out_hbm,
               sidx_all, d0, d1, d2, r0, r1, r2,
               acc_sh, g0, g1, g2, ss0, ss1, ss2):
    c = lax.axis_index("c")
    s = lax.axis_index("s")

    pltpu.sync_copy(zer_hbm, acc_sh.at[pl.ds(s * RPS, RPS)])
    plsc.subcore_barrier()

    rows = (r0, r1, r2)
    didx = (d0, d1, d2)
    gs = (g0, g1, g2)
    ss = (ss0, ss1, ss2)
    n = PCPS
    cbase = s * n

    def gsl(j):
        return sidx_all.at[pl.ds(j * PCH, PCH)]

    pltpu.sync_copy(src2_hbm.at[pl.ds(c * E + cbase * PCH, n * PCH)],
                    sidx_all)
    for u in range(2):
        pltpu.sync_copy(dst_hbm.at[pl.ds((cbase + u) * PCH, PCH)], didx[u])
        pltpu.async_copy(g_hbm.at[gsl(u)], rows[u], gs[u])

    def chunk_step(j, u):
        pv = (u + 2) % 3
        pltpu.make_async_copy(g_hbm.at[gsl(j)], rows[u], gs[u]).wait()
        pltpu.async_copy(rows[u], acc_sh.at[didx[u]], ss[u], add=True)

        @pl.when(j + 2 < n)
        def _(j=j, u=u, pv=pv):
            @pl.when(j >= 1)
            def _():
                pltpu.make_async_copy(
                    rows[pv], acc_sh.at[didx[pv]], ss[pv]).wait()

            pltpu.sync_copy(
                dst_hbm.at[pl.ds((cbase + j + 2) * PCH, PCH)], didx[pv])
            pltpu.async_copy(g_hbm.at[gsl(j + 2)], rows[pv], gs[pv])

    @pl.loop(0, n - n % 3, step=3)
    def _(i):
        for u in range(3):
            chunk_step(i + u, u)

    for j in range(n - n % 3, n):
        chunk_step(j, j % 3)

    for u in range(3):
        pltpu.make_async_copy(rows[u], acc_sh.at[didx[u]], ss[u]).wait()

    plsc.subcore_barrier()

    pltpu.sync_copy(acc_sh.at[pl.ds(s * RPS, RPS)],
                    out_hbm.at[pl.ds(c * NP + s * RPS, RPS)])


def _prop(g, src2, dst, zer):
    k = functools.partial(
        pl.kernel,
        out_type=jax.ShapeDtypeStruct((NC * NP, H), jnp.float32),
        mesh=_sc_mesh(),
        scratch_types=[
            pltpu.VMEM((PCPS * PCH,), jnp.int32),
            pltpu.VMEM((PCH,), jnp.int32),
            pltpu.VMEM((PCH,), jnp.int32),
            pltpu.VMEM((PCH,), jnp.int32),
            pltpu.VMEM((PCH, H), jnp.float32),
            pltpu.VMEM((PCH, H), jnp.float32),
            pltpu.VMEM((PCH, H), jnp.float32),
            pltpu.VMEM_SHARED((NP, H), jnp.float32),
            pltpu.SemaphoreType.DMA,
            pltpu.SemaphoreType.DMA,
            pltpu.SemaphoreType.DMA,
            pltpu.SemaphoreType.DMA,
            pltpu.SemaphoreType.DMA,
            pltpu.SemaphoreType.DMA,
        ],
    )(_prop_body)
    return k(g, src2, dst, zer)


def _mm1_body(h_ref, w_ref, deg0_ref, deg1_ref, out_ref):
    kk = pl.program_id(2)
    part = jnp.dot(h_ref[...], w_ref[...], preferred_element_type=jnp.float32)

    @pl.when(kk == 0)
    def _():
        out_ref[...] = part

    @pl.when(kk == 1)
    def _():
        dinv = lax.rsqrt(deg0_ref[...] + deg1_ref[...] + 1.0)
        out_ref[...] = (out_ref[...] + part) * dinv


def _mm1(h_split, w, deg2d):
    return pl.pallas_call(
        _mm1_body,
        grid=(MB, NC, NC),
        in_specs=[
            pl.BlockSpec((BM, H), lambda m, n, k: (k * MB + m, 0)),
            pl.BlockSpec((H, H), lambda m, n, k: (k, n)),
            pl.BlockSpec((BM, 1), lambda m, n, k: (m, 0)),
            pl.BlockSpec((BM, 1), lambda m, n, k: (MB + m, 0)),
        ],
        out_specs=pl.BlockSpec((BM, H), lambda m, n, k: (n * MB + m, 0)),
        out_shape=jax.ShapeDtypeStruct((NC * NP, H), jnp.float32),
        compiler_params=pltpu.CompilerParams(
            dimension_semantics=("parallel", "parallel", "arbitrary")),
    )(h_split, w, deg2d, deg2d)


def _mmf_body(acc_ref, g_ref, b_ref, w_ref, deg0_ref, deg1_ref, out_ref):
    kk = pl.program_id(2)
    dinv = lax.rsqrt(deg0_ref[...] + deg1_ref[...] + 1.0)
    h = jnp.maximum(
        dinv * (acc_ref[...] + g_ref[...]) + b_ref[0], 0.0)
    part = jnp.dot(h, w_ref[...], preferred_element_type=jnp.float32)

    @pl.when(kk == 0)
    def _():
        out_ref[...] = part

    @pl.when(kk == 1)
    def _():
        out_ref[...] = (out_ref[...] + part) * dinv


def _mmf(acc, g, b3d, w, deg2d):
    return pl.pallas_call(
        _mmf_body,
        grid=(MB, NC, NC),
        in_specs=[
            pl.BlockSpec((BM, H), lambda m, n, k: (k * MB + m, 0)),
            pl.BlockSpec((BM, H), lambda m, n, k: (k * MB + m, 0)),
            pl.BlockSpec((1, 1, H), lambda m, n, k: (k, 0, 0)),
            pl.BlockSpec((H, H), lambda m, n, k: (k, n)),
            pl.BlockSpec((BM, 1), lambda m, n, k: (m, 0)),
            pl.BlockSpec((BM, 1), lambda m, n, k: (MB + m, 0)),
        ],
        out_specs=pl.BlockSpec((BM, H), lambda m, n, k: (n * MB + m, 0)),
        out_shape=jax.ShapeDtypeStruct((NC * NP, H), jnp.float32),
        compiler_params=pltpu.CompilerParams(
            dimension_semantics=("parallel", "parallel", "arbitrary")),
    )(acc, g, b3d, w, deg2d, deg2d)


def _final_body(acc0_ref, g0_ref, acc1_ref, g1_ref, deg0_ref, deg1_ref,
                b_ref, out_ref):
    dinv = lax.rsqrt(deg0_ref[...] + deg1_ref[...] + 1.0)
    p0 = dinv * (acc0_ref[...] + g0_ref[...]) + b_ref[:, :H]
    p1 = dinv * (acc1_ref[...] + g1_ref[...]) + b_ref[:, H:]
    out_ref[...] = jnp.concatenate([p0, p1], axis=1)


def _final(acc, g, deg2d, b2d):
    return pl.pallas_call(
        _final_body,
        grid=(MB,),
        in_specs=[
            pl.BlockSpec((BM, H), lambda m: (m, 0)),
            pl.BlockSpec((BM, H), lambda m: (m, 0)),
            pl.BlockSpec((BM, H), lambda m: (MB + m, 0)),
            pl.BlockSpec((BM, H), lambda m: (MB + m, 0)),
            pl.BlockSpec((BM, 1), lambda m: (m, 0)),
            pl.BlockSpec((BM, 1), lambda m: (MB + m, 0)),
            pl.BlockSpec((1, D), lambda m: (0, 0)),
        ],
        out_specs=pl.BlockSpec((BM, D), lambda m: (m, 0)),
        out_shape=jax.ShapeDtypeStruct((NP, D), jnp.float32),
        compiler_params=pltpu.CompilerParams(
            dimension_semantics=("parallel",)),
    )(acc, g, acc, g, deg2d, deg2d, b2d)


def kernel(x, edge_index, W1, b1, W2, b2, W3, b3):
    src = edge_index[0]
    dst = edge_index[1]
    xp = jnp.pad(x, ((0, NP - N), (0, 0)))
    xs = xp.reshape(NP, NC, H).transpose(1, 0, 2).reshape(NC * NP, H)
    zer = jnp.zeros((RPS, H), jnp.float32)
    src2 = jnp.concatenate([src, src + NP])

    deg2d = _deg(dst).reshape(NC * NP, 1)

    g = _mm1(xs, W1, deg2d)
    acc = _prop(g, src2, dst, zer)
    g = _mmf(acc, g, b1.reshape(NC, 1, H), W2, deg2d)
    acc = _prop(g, src2, dst, zer)
    g = _mmf(acc, g, b2.reshape(NC, 1, H), W3, deg2d)
    acc = _prop(g, src2, dst, zer)

    return _final(acc, g, deg2d, b3.reshape(1, D))[:N]


# speedup vs baseline: 1.1233x; 1.1233x over previous
import functools

import jax
import jax.numpy as jnp
from jax import lax
from jax.experimental import pallas as pl
from jax.experimental.pallas import tpu as pltpu
from jax.experimental.pallas import tpu_sc as plsc

N = 10000
E = 160000
D = 256
H = 128
NP = 10240
NC = 2
NS = 16
ECH = 128
NCH = E // ECH
CPS = NCH // NS
NXTRA = NCH - CPS * NS
RPS = NP // NS
BM = 1024
MB = NP // BM
CPC = NCH // NC
DCPS = CPC // NS


def _sc_mesh():
    return plsc.VectorSubcoreMesh(
        core_axis_name="c", subcore_axis_name="s",
        num_cores=NC, num_subcores=NS)


def _deg(dst):
    @functools.partial(
        pl.kernel,
        out_type=jax.ShapeDtypeStruct((NC * NP,), jnp.float32),
        mesh=_sc_mesh(),
        scratch_types=[
            pltpu.VMEM((ECH,), jnp.int32),
            pltpu.VMEM((ECH,), jnp.int32),
            pltpu.VMEM((ECH,), jnp.float32),
            pltpu.VMEM((RPS,), jnp.float32),
            pltpu.VMEM_SHARED((NP,), jnp.float32),
            pltpu.SemaphoreType.DMA,
            pltpu.SemaphoreType.DMA,
        ],
    )
    def k(dst_hbm, deg_hbm, didx0, didx1, ones_v, val_v, deg_sh, sem0, sem1):
        c = lax.axis_index("c")
        s = lax.axis_index("s")

        @pl.loop(0, RPS // 16)
        def _(i):
            val_v[pl.ds(i * 16, 16)] = jnp.zeros((16,), jnp.float32)

        pltpu.sync_copy(val_v, deg_sh.at[pl.ds(s * RPS, RPS)])

        @pl.loop(0, ECH // 16)
        def _(i):
            ones_v[pl.ds(i * 16, 16)] = jnp.ones((16,), jnp.float32)

        plsc.subcore_barrier()

        ebase = (c * CPC + s * DCPS) * ECH
        pltpu.sync_copy(dst_hbm.at[pl.ds(ebase, ECH)], didx0)

        @pl.loop(0, DCPS, step=2)
        def _(i):
            pltpu.async_copy(ones_v, deg_sh.at[didx0], sem0, add=True)

            @pl.when(i + 1 < DCPS)
            def _():
                pltpu.sync_copy(
                    dst_hbm.at[pl.ds(ebase + (i + 1) * ECH, ECH)], didx1)

            pltpu.make_async_copy(ones_v, deg_sh.at[didx0], sem0).wait()

            @pl.when(i + 1 < DCPS)
            def _():
                pltpu.async_copy(ones_v, deg_sh.at[didx1], sem1, add=True)

                @pl.when(i + 2 < DCPS)
                def _():
                    pltpu.sync_copy(
                        dst_hbm.at[pl.ds(ebase + (i + 2) * ECH, ECH)], didx0)

                pltpu.make_async_copy(ones_v, deg_sh.at[didx1], sem1).wait()

        @pl.when(s == 0)
        def _():
            pltpu.sync_copy(
                dst_hbm.at[pl.ds((c * CPC + NS * DCPS) * ECH, ECH)], didx0)
            pltpu.sync_copy(ones_v, deg_sh.at[didx0], add=True)

        plsc.subcore_barrier()
        pltpu.sync_copy(deg_sh.at[pl.ds(s * RPS, RPS)],
                        deg_hbm.at[pl.ds(c * NP + s * RPS, RPS)])

    return k(dst)


def _prop_body(g_hbm, src2_hbm, dst_hbm, zer_hbm, out_hbm,
               sidx_all, didx0, didx1, r0, r1, acc_sh, s0, s1):
    c = lax.axis_index("c")
    s = lax.axis_index("s")

    pltpu.sync_copy(zer_hbm, acc_sh.at[pl.ds(s * RPS, RPS)])
    plsc.subcore_barrier()

    def gslice(j):
        return sidx_all.at[pl.ds(j * ECH, ECH)]

    def ring(cbase, n):
        pltpu.sync_copy(src2_hbm.at[pl.ds(c * E + cbase * ECH, n * ECH)],
                        sidx_all.at[pl.ds(0, n * ECH)])
        pltpu.sync_copy(dst_hbm.at[pl.ds(cbase * ECH, ECH)], didx0)
        pltpu.async_copy(g_hbm.at[gslice(0)], r0, s0)

        @pl.loop(0, n - n % 2, step=2)
        def _(i):
            @pl.when(i + 1 < n)
            def _():
                pltpu.async_copy(g_hbm.at[gslice(i + 1)], r1, s1)
                pltpu.sync_copy(
                    dst_hbm.at[pl.ds((cbase + i + 1) * ECH, ECH)], didx1)

            pltpu.make_async_copy(g_hbm.at[gslice(i)], r0, s0).wait()
            pltpu.sync_copy(r0, acc_sh.at[didx0], add=True)

            @pl.when(i + 2 < n)
            def _():
                pltpu.async_copy(g_hbm.at[gslice(i + 2)], r0, s0)
                pltpu.sync_copy(
                    dst_hbm.at[pl.ds((cbase + i + 2) * ECH, ECH)], didx0)

            pltpu.make_async_copy(g_hbm.at[gslice(i + 1)], r1, s1).wait()
            pltpu.sync_copy(r1, acc_sh.at[didx1], add=True)

        if n % 2:
            pltpu.make_async_copy(g_hbm.at[gslice(n - 1)], r0, s0).wait()
            pltpu.sync_copy(r0, acc_sh.at[didx0], add=True)

    @pl.when(s < NXTRA)
    def _():
        ring(s * (CPS + 1), CPS + 1)

    @pl.when(s >= NXTRA)
    def _():
        ring(NXTRA * (CPS + 1) + (s - NXTRA) * CPS, CPS)

    plsc.subcore_barrier()

    pltpu.sync_copy(acc_sh.at[pl.ds(s * RPS, RPS)],
                    out_hbm.at[pl.ds(c * NP + s * RPS, RPS)])


def _prop(g, src2, dst, zer):
    k = functools.partial(
        pl.kernel,
        out_type=jax.ShapeDtypeStruct((NC * NP, H), jnp.float32),
        mesh=_sc_mesh(),
        scratch_types=[
            pltpu.VMEM(((CPS + 1) * ECH,), jnp.int32),
            pltpu.VMEM((ECH,), jnp.int32),
            pltpu.VMEM((ECH,), jnp.int32),
            pltpu.VMEM((ECH, H), jnp.float32),
            pltpu.VMEM((ECH, H), jnp.float32),
            pltpu.VMEM_SHARED((NP, H), jnp.float32),
            pltpu.SemaphoreType.DMA,
            pltpu.SemaphoreType.DMA,
        ],
    )(_prop_body)
    return k(g, src2, dst, zer)


def _mm1_body(x_ref, w_ref, deg0_ref, deg1_ref, out_ref):
    nn = pl.program_id(1)
    dinv = lax.rsqrt(deg0_ref[...] + deg1_ref[...] + 1.0)
    out_ref[...] = dinv * jnp.dot(
        x_ref[...], w_ref[...], preferred_element_type=jnp.float32)


def _mm1(xp, w, deg2d):
    return pl.pallas_call(
        _mm1_body,
        grid=(MB, NC),
        in_specs=[
            pl.BlockSpec((BM, D), lambda m, n: (m, 0)),
            pl.BlockSpec((D, H), lambda m, n: (0, n)),
            pl.BlockSpec((BM, 1), lambda m, n: (m, 0)),
            pl.BlockSpec((BM, 1), lambda m, n: (MB + m, 0)),
        ],
        out_specs=pl.BlockSpec((BM, H), lambda m, n: (n * MB + m, 0)),
        out_shape=jax.ShapeDtypeStruct((NC * NP, H), jnp.float32),
        compiler_params=pltpu.CompilerParams(
            dimension_semantics=("parallel", "arbitrary")),
    )(xp, w, deg2d, deg2d)


def _mmf_body(a0_ref, g0_ref, a1_ref, g1_ref, b_ref, w_ref,
              deg0_ref, deg1_ref, out_ref, h_ref):
    nn = pl.program_id(1)
    dinv = lax.rsqrt(deg0_ref[...] + deg1_ref[...] + 1.0)

    @pl.when(nn == 0)
    def _():
        h0 = dinv * (a0_ref[...] + g0_ref[...]) + b_ref[:, :H]
        h1 = dinv * (a1_ref[...] + g1_ref[...]) + b_ref[:, H:]
        h_ref[...] = jnp.maximum(jnp.concatenate([h0, h1], axis=1), 0.0)

    out_ref[...] = dinv * jnp.dot(
        h_ref[...], w_ref[...], preferred_element_type=jnp.float32)


def _mmf(acc, g, b2d, w, deg2d):
    return pl.pallas_call(
        _mmf_body,
        grid=(MB, NC),
        in_specs=[
            pl.BlockSpec((BM, H), lambda m, n: (m, 0)),
            pl.BlockSpec((BM, H), lambda m, n: (m, 0)),
            pl.BlockSpec((BM, H), lambda m, n: (MB + m, 0)),
            pl.BlockSpec((BM, H), lambda m, n: (MB + m, 0)),
            pl.BlockSpec((1, D), lambda m, n: (0, 0)),
            pl.BlockSpec((D, H), lambda m, n: (0, n)),
            pl.BlockSpec((BM, 1), lambda m, n: (m, 0)),
            pl.BlockSpec((BM, 1), lambda m, n: (MB + m, 0)),
        ],
        out_specs=pl.BlockSpec((BM, H), lambda m, n: (n * MB + m, 0)),
        out_shape=jax.ShapeDtypeStruct((NC * NP, H), jnp.float32),
        scratch_shapes=[pltpu.VMEM((BM, D), jnp.float32)],
        compiler_params=pltpu.CompilerParams(
            dimension_semantics=("arbitrary", "arbitrary")),
    )(acc, g, acc, g, b2d, w, deg2d, deg2d)


def _final_body(a0_ref, g0_ref, a1_ref, g1_ref, deg0_ref, deg1_ref,
                b_ref, out_ref):
    dinv = lax.rsqrt(deg0_ref[...] + deg1_ref[...] + 1.0)
    p0 = dinv * (a0_ref[...] + g0_ref[...]) + b_ref[:, :H]
    p1 = dinv * (a1_ref[...] + g1_ref[...]) + b_ref[:, H:]
    out_ref[...] = jnp.concatenate([p0, p1], axis=1)


def _final(acc, g, deg2d, b2d):
    return pl.pallas_call(
        _final_body,
        grid=(MB,),
        in_specs=[
            pl.BlockSpec((BM, H), lambda m: (m, 0)),
            pl.BlockSpec((BM, H), lambda m: (m, 0)),
            pl.BlockSpec((BM, H), lambda m: (MB + m, 0)),
            pl.BlockSpec((BM, H), lambda m: (MB + m, 0)),
            pl.BlockSpec((BM, 1), lambda m: (m, 0)),
            pl.BlockSpec((BM, 1), lambda m: (MB + m, 0)),
            pl.BlockSpec((1, D), lambda m: (0, 0)),
        ],
        out_specs=pl.BlockSpec((BM, D), lambda m: (m, 0)),
        out_shape=jax.ShapeDtypeStruct((NP, D), jnp.float32),
        compiler_params=pltpu.CompilerParams(
            dimension_semantics=("parallel",)),
    )(acc, g, acc, g, deg2d, deg2d, b2d)


def kernel(x, edge_index, W1, b1, W2, b2, W3, b3):
    src = edge_index[0]
    dst = edge_index[1]
    xp = jnp.pad(x, ((0, NP - N), (0, 0)))
    zer = jnp.zeros((RPS, H), jnp.float32)
    src2 = jnp.concatenate([src, src + NP])

    deg2d = _deg(dst).reshape(NC * NP, 1)

    g = _mm1(xp, W1, deg2d)
    acc = _prop(g, src2, dst, zer)
    g = _mmf(acc, g, b1.reshape(1, D), W2, deg2d)
    acc = _prop(g, src2, dst, zer)
    g = _mmf(acc, g, b2.reshape(1, D), W3, deg2d)
    acc = _prop(g, src2, dst, zer)

    return _final(acc, g, deg2d, b3.reshape(1, D))[:N]


# speedup vs baseline: 1.1940x; 1.0629x over previous
import functools

import jax
import jax.numpy as jnp
from jax import lax
from jax.experimental import pallas as pl
from jax.experimental.pallas import tpu as pltpu
from jax.experimental.pallas import tpu_sc as plsc

N = 10000
E = 160000
D = 256
H = 128
NP = 10240
NC = 2
NS = 16
ECH = 128
NCH = E // ECH
CPS = NCH // NS
NXTRA = NCH - CPS * NS
RPS = NP // NS
BM = 2000
NBM = N // BM
CPC = NCH // NC
DCPS = CPC // NS


def _sc_mesh():
    return plsc.VectorSubcoreMesh(
        core_axis_name="c", subcore_axis_name="s",
        num_cores=NC, num_subcores=NS)


def _deg(dst):
    @functools.partial(
        pl.kernel,
        out_type=jax.ShapeDtypeStruct((NC * NP,), jnp.float32),
        mesh=_sc_mesh(),
        scratch_types=[
            pltpu.VMEM((ECH,), jnp.int32),
            pltpu.VMEM((ECH,), jnp.int32),
            pltpu.VMEM((ECH,), jnp.float32),
            pltpu.VMEM((RPS,), jnp.float32),
            pltpu.VMEM_SHARED((NP,), jnp.float32),
            pltpu.SemaphoreType.DMA,
            pltpu.SemaphoreType.DMA,
        ],
    )
    def k(dst_hbm, deg_hbm, didx0, didx1, ones_v, val_v, deg_sh, sem0, sem1):
        c = lax.axis_index("c")
        s = lax.axis_index("s")

        @pl.loop(0, RPS // 16)
        def _(i):
            val_v[pl.ds(i * 16, 16)] = jnp.zeros((16,), jnp.float32)

        pltpu.sync_copy(val_v, deg_sh.at[pl.ds(s * RPS, RPS)])

        @pl.loop(0, ECH // 16)
        def _(i):
            ones_v[pl.ds(i * 16, 16)] = jnp.ones((16,), jnp.float32)

        plsc.subcore_barrier()

        ebase = (c * CPC + s * DCPS) * ECH
        pltpu.sync_copy(dst_hbm.at[pl.ds(ebase, ECH)], didx0)

        @pl.loop(0, DCPS, step=2)
        def _(i):
            pltpu.async_copy(ones_v, deg_sh.at[didx0], sem0, add=True)

            @pl.when(i + 1 < DCPS)
            def _():
                pltpu.sync_copy(
                    dst_hbm.at[pl.ds(ebase + (i + 1) * ECH, ECH)], didx1)

            pltpu.make_async_copy(ones_v, deg_sh.at[didx0], sem0).wait()

            @pl.when(i + 1 < DCPS)
            def _():
                pltpu.async_copy(ones_v, deg_sh.at[didx1], sem1, add=True)

                @pl.when(i + 2 < DCPS)
                def _():
                    pltpu.sync_copy(
                        dst_hbm.at[pl.ds(ebase + (i + 2) * ECH, ECH)], didx0)

                pltpu.make_async_copy(ones_v, deg_sh.at[didx1], sem1).wait()

        @pl.when(s == 0)
        def _():
            pltpu.sync_copy(
                dst_hbm.at[pl.ds((c * CPC + NS * DCPS) * ECH, ECH)], didx0)
            pltpu.sync_copy(ones_v, deg_sh.at[didx0], add=True)

        plsc.subcore_barrier()
        pltpu.sync_copy(deg_sh.at[pl.ds(s * RPS, RPS)],
                        deg_hbm.at[pl.ds(c * NP + s * RPS, RPS)])

    return k(dst)


def _prop_body(g_hbm, src2_hbm, dst_hbm, zer_hbm, out_hbm,
               sidx_all, didx0, didx1, r0, r1, acc_sh, s0, s1):
    c = lax.axis_index("c")
    s = lax.axis_index("s")

    pltpu.sync_copy(zer_hbm, acc_sh.at[pl.ds(s * RPS, RPS)])
    plsc.subcore_barrier()

    def gslice(j):
        return sidx_all.at[pl.ds(j * ECH, ECH)]

    def ring(cbase, n):
        pltpu.sync_copy(src2_hbm.at[pl.ds(c * E + cbase * ECH, n * ECH)],
                        sidx_all.at[pl.ds(0, n * ECH)])
        pltpu.sync_copy(dst_hbm.at[pl.ds(cbase * ECH, ECH)], didx0)
        pltpu.async_copy(g_hbm.at[gslice(0)], r0, s0)

        @pl.loop(0, n - n % 2, step=2)
        def _(i):
            @pl.when(i + 1 < n)
            def _():
                pltpu.async_copy(g_hbm.at[gslice(i + 1)], r1, s1)
                pltpu.sync_copy(
                    dst_hbm.at[pl.ds((cbase + i + 1) * ECH, ECH)], didx1)

            pltpu.make_async_copy(g_hbm.at[gslice(i)], r0, s0).wait()
            pltpu.sync_copy(r0, acc_sh.at[didx0], add=True)

            @pl.when(i + 2 < n)
            def _():
                pltpu.async_copy(g_hbm.at[gslice(i + 2)], r0, s0)
                pltpu.sync_copy(
                    dst_hbm.at[pl.ds((cbase + i + 2) * ECH, ECH)], didx0)

            pltpu.make_async_copy(g_hbm.at[gslice(i + 1)], r1, s1).wait()
            pltpu.sync_copy(r1, acc_sh.at[didx1], add=True)

        if n % 2:
            pltpu.make_async_copy(g_hbm.at[gslice(n - 1)], r0, s0).wait()
            pltpu.sync_copy(r0, acc_sh.at[didx0], add=True)

    @pl.when(s < NXTRA)
    def _():
        ring(s * (CPS + 1), CPS + 1)

    @pl.when(s >= NXTRA)
    def _():
        ring(NXTRA * (CPS + 1) + (s - NXTRA) * CPS, CPS)

    plsc.subcore_barrier()

    pltpu.sync_copy(acc_sh.at[pl.ds(s * RPS, RPS)],
                    out_hbm.at[pl.ds(c * NP + s * RPS, RPS)])


def _prop(g, src2, dst, zer):
    k = functools.partial(
        pl.kernel,
        out_type=jax.ShapeDtypeStruct((NC * NP, H), jnp.float32),
        mesh=_sc_mesh(),
        scratch_types=[
            pltpu.VMEM(((CPS + 1) * ECH,), jnp.int32),
            pltpu.VMEM((ECH,), jnp.int32),
            pltpu.VMEM((ECH,), jnp.int32),
            pltpu.VMEM((ECH, H), jnp.float32),
            pltpu.VMEM((ECH, H), jnp.float32),
            pltpu.VMEM_SHARED((NP, H), jnp.float32),
            pltpu.SemaphoreType.DMA,
            pltpu.SemaphoreType.DMA,
        ],
    )(_prop_body)
    return k(g, src2, dst, zer)


def _mm1_body(x_ref, w_ref, deg0_ref, deg1_ref, out_ref):
    dinv = lax.rsqrt(deg0_ref[...] + deg1_ref[...] + 1.0)
    out_ref[...] = dinv * jnp.dot(
        x_ref[...].astype(jnp.bfloat16), w_ref[...].astype(jnp.bfloat16),
        preferred_element_type=jnp.float32)[None]


def _mm1(x, w, deg3):
    return pl.pallas_call(
        _mm1_body,
        grid=(NBM, NC),
        in_specs=[
            pl.BlockSpec((BM, D), lambda m, n: (m, 0)),
            pl.BlockSpec((D, H), lambda m, n: (0, n)),
            pl.BlockSpec((1, BM, 1), lambda m, n: (0, m, 0)),
            pl.BlockSpec((1, BM, 1), lambda m, n: (1, m, 0)),
        ],
        out_specs=pl.BlockSpec((1, BM, H), lambda m, n: (n, m, 0)),
        out_shape=jax.ShapeDtypeStruct((NC, NP, H), jnp.float32),
        compiler_params=pltpu.CompilerParams(
            dimension_semantics=("parallel", "arbitrary")),
    )(x, w, deg3, deg3)


def _mmf_body(a0_ref, g0_ref, a1_ref, g1_ref, b_ref, w_ref,
              deg0_ref, deg1_ref, out_ref, h_ref):
    nn = pl.program_id(1)
    dinv = lax.rsqrt(deg0_ref[...] + deg1_ref[...] + 1.0)

    @pl.when(nn == 0)
    def _():
        h0 = dinv[0] * (a0_ref[0] + g0_ref[0]) + b_ref[:, :H]
        h1 = dinv[0] * (a1_ref[0] + g1_ref[0]) + b_ref[:, H:]
        h_ref[...] = jnp.maximum(
            jnp.concatenate([h0, h1], axis=1), 0.0).astype(jnp.bfloat16)

    out_ref[...] = dinv * jnp.dot(
        h_ref[...], w_ref[...].astype(jnp.bfloat16),
        preferred_element_type=jnp.float32)[None]


def _mmf(acc3, g3, b2d, w, deg3):
    return pl.pallas_call(
        _mmf_body,
        grid=(NBM, NC),
        in_specs=[
            pl.BlockSpec((1, BM, H), lambda m, n: (0, m, 0)),
            pl.BlockSpec((1, BM, H), lambda m, n: (0, m, 0)),
            pl.BlockSpec((1, BM, H), lambda m, n: (1, m, 0)),
            pl.BlockSpec((1, BM, H), lambda m, n: (1, m, 0)),
            pl.BlockSpec((1, D), lambda m, n: (0, 0)),
            pl.BlockSpec((D, H), lambda m, n: (0, n)),
            pl.BlockSpec((1, BM, 1), lambda m, n: (0, m, 0)),
            pl.BlockSpec((1, BM, 1), lambda m, n: (1, m, 0)),
        ],
        out_specs=pl.BlockSpec((1, BM, H), lambda m, n: (n, m, 0)),
        out_shape=jax.ShapeDtypeStruct((NC, NP, H), jnp.float32),
        scratch_shapes=[pltpu.VMEM((BM, D), jnp.bfloat16)],
        compiler_params=pltpu.CompilerParams(
            dimension_semantics=("arbitrary", "arbitrary")),
    )(acc3, g3, acc3, g3, b2d, w, deg3, deg3)


def _final_body(a0_ref, g0_ref, a1_ref, g1_ref, deg0_ref, deg1_ref,
                b_ref, out_ref):
    dinv = lax.rsqrt(deg0_ref[0] + deg1_ref[0] + 1.0)
    p0 = dinv * (a0_ref[0] + g0_ref[0]) + b_ref[:, :H]
    p1 = dinv * (a1_ref[0] + g1_ref[0]) + b_ref[:, H:]
    out_ref[...] = jnp.concatenate([p0, p1], axis=1)


def _final(acc3, g3, deg3, b2d):
    return pl.pallas_call(
        _final_body,
        grid=(NBM,),
        in_specs=[
            pl.BlockSpec((1, BM, H), lambda m: (0, m, 0)),
            pl.BlockSpec((1, BM, H), lambda m: (0, m, 0)),
            pl.BlockSpec((1, BM, H), lambda m: (1, m, 0)),
            pl.BlockSpec((1, BM, H), lambda m: (1, m, 0)),
            pl.BlockSpec((1, BM, 1), lambda m: (0, m, 0)),
            pl.BlockSpec((1, BM, 1), lambda m: (1, m, 0)),
            pl.BlockSpec((1, D), lambda m: (0, 0)),
        ],
        out_specs=pl.BlockSpec((BM, D), lambda m: (m, 0)),
        out_shape=jax.ShapeDtypeStruct((N, D), jnp.float32),
        compiler_params=pltpu.CompilerParams(
            dimension_semantics=("parallel",)),
    )(acc3, g3, acc3, g3, deg3, deg3, b2d)


def kernel(x, edge_index, W1, b1, W2, b2, W3, b3):
    src = edge_index[0]
    dst = edge_index[1]
    zer = jnp.zeros((RPS, H), jnp.float32)
    src2 = jnp.concatenate([src, src + NP])

    deg3 = _deg(dst).reshape(NC, NP, 1)

    g3 = _mm1(x, W1, deg3)
    acc3 = _prop(g3.reshape(NC * NP, H), src2, dst, zer).reshape(NC, NP, H)
    g3 = _mmf(acc3, g3, b1.reshape(1, D), W2, deg3)
    acc3 = _prop(g3.reshape(NC * NP, H), src2, dst, zer).reshape(NC, NP, H)
    g3 = _mmf(acc3, g3, b2.reshape(1, D), W3, deg3)
    acc3 = _prop(g3.reshape(NC * NP, H), src2, dst, zer).reshape(NC, NP, H)

    return _final(acc3, g3, deg3, b3.reshape(1, D))


# speedup vs baseline: 1.2328x; 1.0325x over previous
import functools

import jax
import jax.numpy as jnp
from jax import lax
from jax.experimental import pallas as pl
from jax.experimental.pallas import tpu as pltpu
from jax.experimental.pallas import tpu_sc as plsc

N = 10000
E = 160000
D = 256
H = 128
NP = 10240
NC = 2
NS = 16
ECH = 128
NCH = E // ECH
CPS = NCH // NS
NXTRA = NCH - CPS * NS
RPS = NP // NS
BM = 5000
NBM = N // BM
CPC = NCH // NC
DCPS = CPC // NS


def _sc_mesh():
    return plsc.VectorSubcoreMesh(
        core_axis_name="c", subcore_axis_name="s",
        num_cores=NC, num_subcores=NS)


def _deg(dst):
    @functools.partial(
        pl.kernel,
        out_type=jax.ShapeDtypeStruct((NC * NP,), jnp.float32),
        mesh=_sc_mesh(),
        scratch_types=[
            pltpu.VMEM((ECH,), jnp.int32),
            pltpu.VMEM((ECH,), jnp.int32),
            pltpu.VMEM((ECH,), jnp.float32),
            pltpu.VMEM((RPS,), jnp.float32),
            pltpu.VMEM_SHARED((NP,), jnp.float32),
            pltpu.SemaphoreType.DMA,
            pltpu.SemaphoreType.DMA,
        ],
    )
    def k(dst_hbm, deg_hbm, didx0, didx1, ones_v, val_v, deg_sh, sem0, sem1):
        c = lax.axis_index("c")
        s = lax.axis_index("s")

        @pl.loop(0, RPS // 16)
        def _(i):
            val_v[pl.ds(i * 16, 16)] = jnp.zeros((16,), jnp.float32)

        pltpu.sync_copy(val_v, deg_sh.at[pl.ds(s * RPS, RPS)])

        @pl.loop(0, ECH // 16)
        def _(i):
            ones_v[pl.ds(i * 16, 16)] = jnp.ones((16,), jnp.float32)

        plsc.subcore_barrier()

        ebase = (c * CPC + s * DCPS) * ECH
        pltpu.sync_copy(dst_hbm.at[pl.ds(ebase, ECH)], didx0)

        @pl.loop(0, DCPS, step=2)
        def _(i):
            pltpu.async_copy(ones_v, deg_sh.at[didx0], sem0, add=True)

            @pl.when(i + 1 < DCPS)
            def _():
                pltpu.sync_copy(
                    dst_hbm.at[pl.ds(ebase + (i + 1) * ECH, ECH)], didx1)

            pltpu.make_async_copy(ones_v, deg_sh.at[didx0], sem0).wait()

            @pl.when(i + 1 < DCPS)
            def _():
                pltpu.async_copy(ones_v, deg_sh.at[didx1], sem1, add=True)

                @pl.when(i + 2 < DCPS)
                def _():
                    pltpu.sync_copy(
                        dst_hbm.at[pl.ds(ebase + (i + 2) * ECH, ECH)], didx0)

                pltpu.make_async_copy(ones_v, deg_sh.at[didx1], sem1).wait()

        @pl.when(s == 0)
        def _():
            pltpu.sync_copy(
                dst_hbm.at[pl.ds((c * CPC + NS * DCPS) * ECH, ECH)], didx0)
            pltpu.sync_copy(ones_v, deg_sh.at[didx0], add=True)

        plsc.subcore_barrier()
        pltpu.sync_copy(deg_sh.at[pl.ds(s * RPS, RPS)],
                        deg_hbm.at[pl.ds(c * NP + s * RPS, RPS)])

    return k(dst)


def _prop_body(g_hbm, src2_hbm, dst_hbm, zer_hbm, out_hbm,
               sidx_all, didx0, didx1, r0, r1, acc_sh, s0, s1):
    c = lax.axis_index("c")
    s = lax.axis_index("s")

    pltpu.sync_copy(zer_hbm, acc_sh.at[pl.ds(s * RPS, RPS)])
    plsc.subcore_barrier()

    def gslice(j):
        return sidx_all.at[pl.ds(j * ECH, ECH)]

    def ring(cbase, n):
        pltpu.sync_copy(src2_hbm.at[pl.ds(c * E + cbase * ECH, n * ECH)],
                        sidx_all.at[pl.ds(0, n * ECH)])
        pltpu.sync_copy(dst_hbm.at[pl.ds(cbase * ECH, ECH)], didx0)
        pltpu.async_copy(g_hbm.at[gslice(0)], r0, s0)

        @pl.loop(0, n - n % 2, step=2)
        def _(i):
            @pl.when(i + 1 < n)
            def _():
                pltpu.async_copy(g_hbm.at[gslice(i + 1)], r1, s1)
                pltpu.sync_copy(
                    dst_hbm.at[pl.ds((cbase + i + 1) * ECH, ECH)], didx1)

            pltpu.make_async_copy(g_hbm.at[gslice(i)], r0, s0).wait()
            pltpu.sync_copy(r0, acc_sh.at[didx0], add=True)

            @pl.when(i + 2 < n)
            def _():
                pltpu.async_copy(g_hbm.at[gslice(i + 2)], r0, s0)
                pltpu.sync_copy(
                    dst_hbm.at[pl.ds((cbase + i + 2) * ECH, ECH)], didx0)

            pltpu.make_async_copy(g_hbm.at[gslice(i + 1)], r1, s1).wait()
            pltpu.sync_copy(r1, acc_sh.at[didx1], add=True)

        if n % 2:
            pltpu.make_async_copy(g_hbm.at[gslice(n - 1)], r0, s0).wait()
            pltpu.sync_copy(r0, acc_sh.at[didx0], add=True)

    @pl.when(s < NXTRA)
    def _():
        ring(s * (CPS + 1), CPS + 1)

    @pl.when(s >= NXTRA)
    def _():
        ring(NXTRA * (CPS + 1) + (s - NXTRA) * CPS, CPS)

    plsc.subcore_barrier()

    pltpu.sync_copy(acc_sh.at[pl.ds(s * RPS, RPS)],
                    out_hbm.at[pl.ds(c * NP + s * RPS, RPS)])


def _prop(g, src2, dst, zer):
    k = functools.partial(
        pl.kernel,
        out_type=jax.ShapeDtypeStruct((NC * NP, H), jnp.float32),
        mesh=_sc_mesh(),
        scratch_types=[
            pltpu.VMEM(((CPS + 1) * ECH,), jnp.int32),
            pltpu.VMEM((ECH,), jnp.int32),
            pltpu.VMEM((ECH,), jnp.int32),
            pltpu.VMEM((ECH, H), jnp.float32),
            pltpu.VMEM((ECH, H), jnp.float32),
            pltpu.VMEM_SHARED((NP, H), jnp.float32),
            pltpu.SemaphoreType.DMA,
            pltpu.SemaphoreType.DMA,
        ],
    )(_prop_body)
    return k(g, src2, dst, zer)


def _mm1_body(x_ref, w_ref, deg0_ref, deg1_ref, out_ref):
    dinv = lax.rsqrt(deg0_ref[...] + deg1_ref[...] + 1.0)
    out_ref[...] = dinv * jnp.dot(
        x_ref[...].astype(jnp.bfloat16), w_ref[...].astype(jnp.bfloat16),
        preferred_element_type=jnp.float32)[None]


def _mm1(x, w, deg3):
    return pl.pallas_call(
        _mm1_body,
        grid=(NBM, NC),
        in_specs=[
            pl.BlockSpec((BM, D), lambda m, n: (m, 0)),
            pl.BlockSpec((D, H), lambda m, n: (0, n)),
            pl.BlockSpec((1, BM, 1), lambda m, n: (0, m, 0)),
            pl.BlockSpec((1, BM, 1), lambda m, n: (1, m, 0)),
        ],
        out_specs=pl.BlockSpec((1, BM, H), lambda m, n: (n, m, 0)),
        out_shape=jax.ShapeDtypeStruct((NC, NP, H), jnp.float32),
        compiler_params=pltpu.CompilerParams(
            dimension_semantics=("parallel", "arbitrary")),
    )(x, w, deg3, deg3)


def _mmf_body(a0_ref, g0_ref, a1_ref, g1_ref, b_ref, w_ref,
              deg0_ref, deg1_ref, out_ref, h_ref):
    nn = pl.program_id(1)
    dinv = lax.rsqrt(deg0_ref[...] + deg1_ref[...] + 1.0)

    @pl.when(nn == 0)
    def _():
        h0 = dinv[0] * (a0_ref[0] + g0_ref[0]) + b_ref[:, :H]
        h1 = dinv[0] * (a1_ref[0] + g1_ref[0]) + b_ref[:, H:]
        h_ref[...] = jnp.maximum(
            jnp.concatenate([h0, h1], axis=1), 0.0).astype(jnp.bfloat16)

    out_ref[...] = dinv * jnp.dot(
        h_ref[...], w_ref[...].astype(jnp.bfloat16),
        preferred_element_type=jnp.float32)[None]


def _mmf(acc3, g3, b2d, w, deg3):
    return pl.pallas_call(
        _mmf_body,
        grid=(NBM, NC),
        in_specs=[
            pl.BlockSpec((1, BM, H), lambda m, n: (0, m, 0)),
            pl.BlockSpec((1, BM, H), lambda m, n: (0, m, 0)),
            pl.BlockSpec((1, BM, H), lambda m, n: (1, m, 0)),
            pl.BlockSpec((1, BM, H), lambda m, n: (1, m, 0)),
            pl.BlockSpec((1, D), lambda m, n: (0, 0)),
            pl.BlockSpec((D, H), lambda m, n: (0, n)),
            pl.BlockSpec((1, BM, 1), lambda m, n: (0, m, 0)),
            pl.BlockSpec((1, BM, 1), lambda m, n: (1, m, 0)),
        ],
        out_specs=pl.BlockSpec((1, BM, H), lambda m, n: (n, m, 0)),
        out_shape=jax.ShapeDtypeStruct((NC, NP, H), jnp.float32),
        scratch_shapes=[pltpu.VMEM((BM, D), jnp.bfloat16)],
        compiler_params=pltpu.CompilerParams(
            dimension_semantics=("arbitrary", "arbitrary")),
    )(acc3, g3, acc3, g3, b2d, w, deg3, deg3)


def _final_body(a0_ref, g0_ref, a1_ref, g1_ref, deg0_ref, deg1_ref,
                b_ref, out_ref):
    dinv = lax.rsqrt(deg0_ref[0] + deg1_ref[0] + 1.0)
    p0 = dinv * (a0_ref[0] + g0_ref[0]) + b_ref[:, :H]
    p1 = dinv * (a1_ref[0] + g1_ref[0]) + b_ref[:, H:]
    out_ref[...] = jnp.concatenate([p0, p1], axis=1)


def _final(acc3, g3, deg3, b2d):
    return pl.pallas_call(
        _final_body,
        grid=(NBM,),
        in_specs=[
            pl.BlockSpec((1, BM, H), lambda m: (0, m, 0)),
            pl.BlockSpec((1, BM, H), lambda m: (0, m, 0)),
            pl.BlockSpec((1, BM, H), lambda m: (1, m, 0)),
            pl.BlockSpec((1, BM, H), lambda m: (1, m, 0)),
            pl.BlockSpec((1, BM, 1), lambda m: (0, m, 0)),
            pl.BlockSpec((1, BM, 1), lambda m: (1, m, 0)),
            pl.BlockSpec((1, D), lambda m: (0, 0)),
        ],
        out_specs=pl.BlockSpec((BM, D), lambda m: (m, 0)),
        out_shape=jax.ShapeDtypeStruct((N, D), jnp.float32),
        compiler_params=pltpu.CompilerParams(
            dimension_semantics=("parallel",)),
    )(acc3, g3, acc3, g3, deg3, deg3, b2d)


def kernel(x, edge_index, W1, b1, W2, b2, W3, b3):
    src = edge_index[0]
    dst = edge_index[1]
    zer = jnp.zeros((RPS, H), jnp.float32)
    src2 = jnp.concatenate([src, src + NP])

    deg3 = _deg(dst).reshape(NC, NP, 1)

    g3 = _mm1(x, W1, deg3)
    acc3 = _prop(g3.reshape(NC * NP, H), src2, dst, zer).reshape(NC, NP, H)
    g3 = _mmf(acc3, g3, b1.reshape(1, D), W2, deg3)
    acc3 = _prop(g3.reshape(NC * NP, H), src2, dst, zer).reshape(NC, NP, H)
    g3 = _mmf(acc3, g3, b2.reshape(1, D), W3, deg3)
    acc3 = _prop(g3.reshape(NC * NP, H), src2, dst, zer).reshape(NC, NP, H)

    return _final(acc3, g3, deg3, b3.reshape(1, D))


# speedup vs baseline: 1.2448x; 1.0097x over previous
import functools

import jax
import jax.numpy as jnp
from jax import lax
from jax.experimental import pallas as pl
from jax.experimental.pallas import tpu as pltpu
from jax.experimental.pallas import tpu_sc as plsc

N = 10000
E = 160000
D = 256
H = 128
NP = 10240
NC = 2
NS = 16
ECH = 128
NCH = E // ECH
CPS = NCH // NS
NXTRA = NCH - CPS * NS
RPS = NP // NS
BM = 10000
NBM = N // BM
CPC = NCH // NC
DCPS = CPC // NS


def _sc_mesh():
    return plsc.VectorSubcoreMesh(
        core_axis_name="c", subcore_axis_name="s",
        num_cores=NC, num_subcores=NS)


def _deg(dst):
    @functools.partial(
        pl.kernel,
        out_type=jax.ShapeDtypeStruct((NC * NP,), jnp.float32),
        mesh=_sc_mesh(),
        scratch_types=[
            pltpu.VMEM((ECH,), jnp.int32),
            pltpu.VMEM((ECH,), jnp.int32),
            pltpu.VMEM((ECH,), jnp.float32),
            pltpu.VMEM((RPS,), jnp.float32),
            pltpu.VMEM_SHARED((NP,), jnp.float32),
            pltpu.SemaphoreType.DMA,
            pltpu.SemaphoreType.DMA,
        ],
    )
    def k(dst_hbm, deg_hbm, didx0, didx1, ones_v, val_v, deg_sh, sem0, sem1):
        c = lax.axis_index("c")
        s = lax.axis_index("s")

        @pl.loop(0, RPS // 16)
        def _(i):
            val_v[pl.ds(i * 16, 16)] = jnp.zeros((16,), jnp.float32)

        pltpu.sync_copy(val_v, deg_sh.at[pl.ds(s * RPS, RPS)])

        @pl.loop(0, ECH // 16)
        def _(i):
            ones_v[pl.ds(i * 16, 16)] = jnp.ones((16,), jnp.float32)

        plsc.subcore_barrier()

        ebase = (c * CPC + s * DCPS) * ECH
        pltpu.sync_copy(dst_hbm.at[pl.ds(ebase, ECH)], didx0)

        @pl.loop(0, DCPS, step=2)
        def _(i):
            pltpu.async_copy(ones_v, deg_sh.at[didx0], sem0, add=True)

            @pl.when(i + 1 < DCPS)
            def _():
                pltpu.sync_copy(
                    dst_hbm.at[pl.ds(ebase + (i + 1) * ECH, ECH)], didx1)

            pltpu.make_async_copy(ones_v, deg_sh.at[didx0], sem0).wait()

            @pl.when(i + 1 < DCPS)
            def _():
                pltpu.async_copy(ones_v, deg_sh.at[didx1], sem1, add=True)

                @pl.when(i + 2 < DCPS)
                def _():
                    pltpu.sync_copy(
                        dst_hbm.at[pl.ds(ebase + (i + 2) * ECH, ECH)], didx0)

                pltpu.make_async_copy(ones_v, deg_sh.at[didx1], sem1).wait()

        @pl.when(s == 0)
        def _():
            pltpu.sync_copy(
                dst_hbm.at[pl.ds((c * CPC + NS * DCPS) * ECH, ECH)], didx0)
            pltpu.sync_copy(ones_v, deg_sh.at[didx0], add=True)

        plsc.subcore_barrier()
        pltpu.sync_copy(deg_sh.at[pl.ds(s * RPS, RPS)],
                        deg_hbm.at[pl.ds(c * NP + s * RPS, RPS)])

    return k(dst)


def _prop_body(g_hbm, src2_hbm, dst_hbm, zer_hbm, out_hbm,
               sidx_all, didx0, didx1, r0, r1, acc_sh, s0, s1):
    c = lax.axis_index("c")
    s = lax.axis_index("s")

    pltpu.sync_copy(zer_hbm, acc_sh.at[pl.ds(s * RPS, RPS)])
    plsc.subcore_barrier()

    def gslice(j):
        return sidx_all.at[pl.ds(j * ECH, ECH)]

    def ring(cbase, n):
        pltpu.sync_copy(src2_hbm.at[pl.ds(c * E + cbase * ECH, n * ECH)],
                        sidx_all.at[pl.ds(0, n * ECH)])
        pltpu.sync_copy(dst_hbm.at[pl.ds(cbase * ECH, ECH)], didx0)
        pltpu.async_copy(g_hbm.at[gslice(0)], r0, s0)

        @pl.loop(0, n - n % 2, step=2)
        def _(i):
            @pl.when(i + 1 < n)
            def _():
                pltpu.async_copy(g_hbm.at[gslice(i + 1)], r1, s1)
                pltpu.sync_copy(
                    dst_hbm.at[pl.ds((cbase + i + 1) * ECH, ECH)], didx1)

            pltpu.make_async_copy(g_hbm.at[gslice(i)], r0, s0).wait()
            pltpu.sync_copy(r0, acc_sh.at[didx0], add=True)

            @pl.when(i + 2 < n)
            def _():
                pltpu.async_copy(g_hbm.at[gslice(i + 2)], r0, s0)
                pltpu.sync_copy(
                    dst_hbm.at[pl.ds((cbase + i + 2) * ECH, ECH)], didx0)

            pltpu.make_async_copy(g_hbm.at[gslice(i + 1)], r1, s1).wait()
            pltpu.sync_copy(r1, acc_sh.at[didx1], add=True)

        if n % 2:
            pltpu.make_async_copy(g_hbm.at[gslice(n - 1)], r0, s0).wait()
            pltpu.sync_copy(r0, acc_sh.at[didx0], add=True)

    @pl.when(s < NXTRA)
    def _():
        ring(s * (CPS + 1), CPS + 1)

    @pl.when(s >= NXTRA)
    def _():
        ring(NXTRA * (CPS + 1) + (s - NXTRA) * CPS, CPS)

    plsc.subcore_barrier()

    pltpu.sync_copy(acc_sh.at[pl.ds(s * RPS, RPS)],
                    out_hbm.at[pl.ds(c * NP + s * RPS, RPS)])


def _prop(g, src2, dst, zer):
    k = functools.partial(
        pl.kernel,
        out_type=jax.ShapeDtypeStruct((NC * NP, H), jnp.float32),
        mesh=_sc_mesh(),
        scratch_types=[
            pltpu.VMEM(((CPS + 1) * ECH,), jnp.int32),
            pltpu.VMEM((ECH,), jnp.int32),
            pltpu.VMEM((ECH,), jnp.int32),
            pltpu.VMEM((ECH, H), jnp.float32),
            pltpu.VMEM((ECH, H), jnp.float32),
            pltpu.VMEM_SHARED((NP, H), jnp.float32),
            pltpu.SemaphoreType.DMA,
            pltpu.SemaphoreType.DMA,
        ],
    )(_prop_body)
    return k(g, src2, dst, zer)


def _mm1_body(x_ref, w_ref, deg0_ref, deg1_ref, out_ref):
    dinv = lax.rsqrt(deg0_ref[...] + deg1_ref[...] + 1.0)
    out_ref[...] = dinv * jnp.dot(
        x_ref[...].astype(jnp.bfloat16), w_ref[...].astype(jnp.bfloat16),
        preferred_element_type=jnp.float32)[None]


def _mm1(x, w, deg3):
    return pl.pallas_call(
        _mm1_body,
        grid=(NBM, NC),
        in_specs=[
            pl.BlockSpec((BM, D), lambda m, n: (m, 0)),
            pl.BlockSpec((D, H), lambda m, n: (0, n)),
            pl.BlockSpec((1, BM, 1), lambda m, n: (0, m, 0)),
            pl.BlockSpec((1, BM, 1), lambda m, n: (1, m, 0)),
        ],
        out_specs=pl.BlockSpec((1, BM, H), lambda m, n: (n, m, 0)),
        out_shape=jax.ShapeDtypeStruct((NC, NP, H), jnp.float32),
        compiler_params=pltpu.CompilerParams(
            dimension_semantics=("parallel", "arbitrary")),
    )(x, w, deg3, deg3)


def _mmf_body(a0_ref, g0_ref, a1_ref, g1_ref, b_ref, w_ref,
              deg0_ref, deg1_ref, out_ref, h_ref):
    nn = pl.program_id(1)
    dinv = lax.rsqrt(deg0_ref[...] + deg1_ref[...] + 1.0)

    @pl.when(nn == 0)
    def _():
        h0 = dinv[0] * (a0_ref[0] + g0_ref[0]) + b_ref[:, :H]
        h1 = dinv[0] * (a1_ref[0] + g1_ref[0]) + b_ref[:, H:]
        h_ref[...] = jnp.maximum(
            jnp.concatenate([h0, h1], axis=1), 0.0).astype(jnp.bfloat16)

    out_ref[...] = dinv * jnp.dot(
        h_ref[...], w_ref[...].astype(jnp.bfloat16),
        preferred_element_type=jnp.float32)[None]


def _mmf(acc3, g3, b2d, w, deg3):
    return pl.pallas_call(
        _mmf_body,
        grid=(NBM, NC),
        in_specs=[
            pl.BlockSpec((1, BM, H), lambda m, n: (0, m, 0)),
            pl.BlockSpec((1, BM, H), lambda m, n: (0, m, 0)),
            pl.BlockSpec((1, BM, H), lambda m, n: (1, m, 0)),
            pl.BlockSpec((1, BM, H), lambda m, n: (1, m, 0)),
            pl.BlockSpec((1, D), lambda m, n: (0, 0)),
            pl.BlockSpec((D, H), lambda m, n: (0, n)),
            pl.BlockSpec((1, BM, 1), lambda m, n: (0, m, 0)),
            pl.BlockSpec((1, BM, 1), lambda m, n: (1, m, 0)),
        ],
        out_specs=pl.BlockSpec((1, BM, H), lambda m, n: (n, m, 0)),
        out_shape=jax.ShapeDtypeStruct((NC, NP, H), jnp.float32),
        scratch_shapes=[pltpu.VMEM((BM, D), jnp.bfloat16)],
        compiler_params=pltpu.CompilerParams(
            dimension_semantics=("arbitrary", "arbitrary")),
    )(acc3, g3, acc3, g3, b2d, w, deg3, deg3)


def _final_body(a0_ref, g0_ref, a1_ref, g1_ref, deg0_ref, deg1_ref,
                b_ref, out_ref):
    dinv = lax.rsqrt(deg0_ref[0] + deg1_ref[0] + 1.0)
    p0 = dinv * (a0_ref[0] + g0_ref[0]) + b_ref[:, :H]
    p1 = dinv * (a1_ref[0] + g1_ref[0]) + b_ref[:, H:]
    out_ref[...] = jnp.concatenate([p0, p1], axis=1)


def _final(acc3, g3, deg3, b2d):
    return pl.pallas_call(
        _final_body,
        grid=(NBM,),
        in_specs=[
            pl.BlockSpec((1, BM, H), lambda m: (0, m, 0)),
            pl.BlockSpec((1, BM, H), lambda m: (0, m, 0)),
            pl.BlockSpec((1, BM, H), lambda m: (1, m, 0)),
            pl.BlockSpec((1, BM, H), lambda m: (1, m, 0)),
            pl.BlockSpec((1, BM, 1), lambda m: (0, m, 0)),
            pl.BlockSpec((1, BM, 1), lambda m: (1, m, 0)),
            pl.BlockSpec((1, D), lambda m: (0, 0)),
        ],
        out_specs=pl.BlockSpec((BM, D), lambda m: (m, 0)),
        out_shape=jax.ShapeDtypeStruct((N, D), jnp.float32),
        compiler_params=pltpu.CompilerParams(
            dimension_semantics=("parallel",)),
    )(acc3, g3, acc3, g3, deg3, deg3, b2d)


def kernel(x, edge_index, W1, b1, W2, b2, W3, b3):
    src = edge_index[0]
    dst = edge_index[1]
    zer = jnp.zeros((RPS, H), jnp.float32)
    src2 = jnp.concatenate([src, src + NP])

    deg3 = _deg(dst).reshape(NC, NP, 1)

    g3 = _mm1(x, W1, deg3)
    acc3 = _prop(g3.reshape(NC * NP, H), src2, dst, zer).reshape(NC, NP, H)
    g3 = _mmf(acc3, g3, b1.reshape(1, D), W2, deg3)
    acc3 = _prop(g3.reshape(NC * NP, H), src2, dst, zer).reshape(NC, NP, H)
    g3 = _mmf(acc3, g3, b2.reshape(1, D), W3, deg3)
    acc3 = _prop(g3.reshape(NC * NP, H), src2, dst, zer).reshape(NC, NP, H)

    return _final(acc3, g3, deg3, b3.reshape(1, D))


# speedup vs baseline: 1.2451x; 1.0002x over previous
import functools

import jax
import jax.numpy as jnp
from jax import lax
from jax.experimental import pallas as pl
from jax.experimental.pallas import tpu as pltpu
from jax.experimental.pallas import tpu_sc as plsc

N = 10000
E = 160000
D = 256
H = 128
NP = 10240
NC = 2
NS = 16
ECH = 128
NCH = E // ECH
CPS = NCH // NS
NXTRA = NCH - CPS * NS
RPS = NP // NS
BM = 10000
NBM = N // BM
CPC = NCH // NC
DCPS = CPC // NS


def _sc_mesh():
    return plsc.VectorSubcoreMesh(
        core_axis_name="c", subcore_axis_name="s",
        num_cores=NC, num_subcores=NS)


def _deg(dst):
    @functools.partial(
        pl.kernel,
        out_type=jax.ShapeDtypeStruct((NC * NP,), jnp.float32),
        mesh=_sc_mesh(),
        scratch_types=[
            pltpu.VMEM((ECH,), jnp.int32),
            pltpu.VMEM((ECH,), jnp.int32),
            pltpu.VMEM((ECH,), jnp.int32),
            pltpu.VMEM((ECH,), jnp.int32),
            pltpu.VMEM((ECH,), jnp.float32),
            pltpu.VMEM((RPS,), jnp.float32),
            pltpu.VMEM_SHARED((NP,), jnp.float32),
            pltpu.SemaphoreType.DMA,
            pltpu.SemaphoreType.DMA,
            pltpu.SemaphoreType.DMA,
            pltpu.SemaphoreType.DMA,
        ],
    )
    def k(dst_hbm, deg_hbm, dx0, dx1, dx2, dx3, ones_v, val_v, deg_sh,
          m0, m1, m2, m3):
        c = lax.axis_index("c")
        s = lax.axis_index("s")
        didx = (dx0, dx1, dx2, dx3)
        sems = (m0, m1, m2, m3)

        @pl.loop(0, RPS // 16)
        def _(i):
            val_v[pl.ds(i * 16, 16)] = jnp.zeros((16,), jnp.float32)

        pltpu.sync_copy(val_v, deg_sh.at[pl.ds(s * RPS, RPS)])

        @pl.loop(0, ECH // 16)
        def _(i):
            ones_v[pl.ds(i * 16, 16)] = jnp.ones((16,), jnp.float32)

        plsc.subcore_barrier()

        ebase = (c * CPC + s * DCPS) * ECH

        def fire(j, u):
            pltpu.sync_copy(dst_hbm.at[pl.ds(ebase + j * ECH, ECH)], didx[u])
            pltpu.async_copy(ones_v, deg_sh.at[didx[u]], sems[u], add=True)

        def drain(u):
            pltpu.make_async_copy(ones_v, deg_sh.at[didx[u]], sems[u]).wait()

        for u in range(4):
            fire(u, u)

        @pl.loop(4, DCPS - (DCPS - 4) % 4, step=4)
        def _(j):
            for u in range(4):
                drain(u)
                fire(j + u, u)

        for j in range(DCPS - (DCPS - 4) % 4, DCPS):
            u = j % 4
            drain(u)
            fire(j, u)

        @pl.when(s == 0)
        def _():
            u = DCPS % 4
            drain(u)
            fire(NS * DCPS, u)

        for u in range(4):
            drain(u)

        plsc.subcore_barrier()
        pltpu.sync_copy(deg_sh.at[pl.ds(s * RPS, RPS)],
                        deg_hbm.at[pl.ds(c * NP + s * RPS, RPS)])

    return k(dst)


def _prop_body(g_hbm, src2_hbm, dst_hbm, zer_hbm, out_hbm,
               sidx_all, didx0, didx1, r0, r1, acc_sh, s0, s1):
    c = lax.axis_index("c")
    s = lax.axis_index("s")

    pltpu.sync_copy(zer_hbm, acc_sh.at[pl.ds(s * RPS, RPS)])
    plsc.subcore_barrier()

    def gslice(j):
        return sidx_all.at[pl.ds(j * ECH, ECH)]

    def ring(cbase, n):
        pltpu.sync_copy(src2_hbm.at[pl.ds(c * E + cbase * ECH, n * ECH)],
                        sidx_all.at[pl.ds(0, n * ECH)])
        pltpu.sync_copy(dst_hbm.at[pl.ds(cbase * ECH, ECH)], didx0)
        pltpu.async_copy(g_hbm.at[gslice(0)], r0, s0)

        @pl.loop(0, n - n % 2, step=2)
        def _(i):
            @pl.when(i + 1 < n)
            def _():
                pltpu.async_copy(g_hbm.at[gslice(i + 1)], r1, s1)
                pltpu.sync_copy(
                    dst_hbm.at[pl.ds((cbase + i + 1) * ECH, ECH)], didx1)

            pltpu.make_async_copy(g_hbm.at[gslice(i)], r0, s0).wait()
            pltpu.sync_copy(r0, acc_sh.at[didx0], add=True)

            @pl.when(i + 2 < n)
            def _():
                pltpu.async_copy(g_hbm.at[gslice(i + 2)], r0, s0)
                pltpu.sync_copy(
                    dst_hbm.at[pl.ds((cbase + i + 2) * ECH, ECH)], didx0)

            pltpu.make_async_copy(g_hbm.at[gslice(i + 1)], r1, s1).wait()
            pltpu.sync_copy(r1, acc_sh.at[didx1], add=True)

        if n % 2:
            pltpu.make_async_copy(g_hbm.at[gslice(n - 1)], r0, s0).wait()
            pltpu.sync_copy(r0, acc_sh.at[didx0], add=True)

    @pl.when(s < NXTRA)
    def _():
        ring(s * (CPS + 1), CPS + 1)

    @pl.when(s >= NXTRA)
    def _():
        ring(NXTRA * (CPS + 1) + (s - NXTRA) * CPS, CPS)

    plsc.subcore_barrier()

    pltpu.sync_copy(acc_sh.at[pl.ds(s * RPS, RPS)],
                    out_hbm.at[pl.ds(c * NP + s * RPS, RPS)])


def _prop(g, src2, dst, zer):
    k = functools.partial(
        pl.kernel,
        out_type=jax.ShapeDtypeStruct((NC * NP, H), jnp.float32),
        mesh=_sc_mesh(),
        scratch_types=[
            pltpu.VMEM(((CPS + 1) * ECH,), jnp.int32),
            pltpu.VMEM((ECH,), jnp.int32),
            pltpu.VMEM((ECH,), jnp.int32),
            pltpu.VMEM((ECH, H), jnp.float32),
            pltpu.VMEM((ECH, H), jnp.float32),
            pltpu.VMEM_SHARED((NP, H), jnp.float32),
            pltpu.SemaphoreType.DMA,
            pltpu.SemaphoreType.DMA,
        ],
    )(_prop_body)
    return k(g, src2, dst, zer)


def _mm1_body(x_ref, w_ref, deg0_ref, deg1_ref, out_ref):
    dinv = lax.rsqrt(deg0_ref[...] + deg1_ref[...] + 1.0)
    out_ref[...] = dinv * jnp.dot(
        x_ref[...].astype(jnp.bfloat16), w_ref[...].astype(jnp.bfloat16),
        preferred_element_type=jnp.float32)[None]


def _mm1(x, w, deg3):
    return pl.pallas_call(
        _mm1_body,
        grid=(NBM, NC),
        in_specs=[
            pl.BlockSpec((BM, D), lambda m, n: (m, 0)),
            pl.BlockSpec((D, H), lambda m, n: (0, n)),
            pl.BlockSpec((1, BM, 1), lambda m, n: (0, m, 0)),
            pl.BlockSpec((1, BM, 1), lambda m, n: (1, m, 0)),
        ],
        out_specs=pl.BlockSpec((1, BM, H), lambda m, n: (n, m, 0)),
        out_shape=jax.ShapeDtypeStruct((NC, NP, H), jnp.float32),
        compiler_params=pltpu.CompilerParams(
            dimension_semantics=("parallel", "arbitrary")),
    )(x, w, deg3, deg3)


def _mmf_body(a0_ref, g0_ref, a1_ref, g1_ref, b_ref, w_ref,
              deg0_ref, deg1_ref, out_ref, h_ref):
    nn = pl.program_id(1)
    dinv = lax.rsqrt(deg0_ref[...] + deg1_ref[...] + 1.0)

    @pl.when(nn == 0)
    def _():
        h0 = dinv[0] * (a0_ref[0] + g0_ref[0]) + b_ref[:, :H]
        h1 = dinv[0] * (a1_ref[0] + g1_ref[0]) + b_ref[:, H:]
        h_ref[...] = jnp.maximum(
            jnp.concatenate([h0, h1], axis=1), 0.0).astype(jnp.bfloat16)

    out_ref[...] = dinv * jnp.dot(
        h_ref[...], w_ref[...].astype(jnp.bfloat16),
        preferred_element_type=jnp.float32)[None]


def _mmf(acc3, g3, b2d, w, deg3):
    return pl.pallas_call(
        _mmf_body,
        grid=(NBM, NC),
        in_specs=[
            pl.BlockSpec((1, BM, H), lambda m, n: (0, m, 0)),
            pl.BlockSpec((1, BM, H), lambda m, n: (0, m, 0)),
            pl.BlockSpec((1, BM, H), lambda m, n: (1, m, 0)),
            pl.BlockSpec((1, BM, H), lambda m, n: (1, m, 0)),
            pl.BlockSpec((1, D), lambda m, n: (0, 0)),
            pl.BlockSpec((D, H), lambda m, n: (0, n)),
            pl.BlockSpec((1, BM, 1), lambda m, n: (0, m, 0)),
            pl.BlockSpec((1, BM, 1), lambda m, n: (1, m, 0)),
        ],
        out_specs=pl.BlockSpec((1, BM, H), lambda m, n: (n, m, 0)),
        out_shape=jax.ShapeDtypeStruct((NC, NP, H), jnp.float32),
        scratch_shapes=[pltpu.VMEM((BM, D), jnp.bfloat16)],
        compiler_params=pltpu.CompilerParams(
            dimension_semantics=("arbitrary", "arbitrary")),
    )(acc3, g3, acc3, g3, b2d, w, deg3, deg3)


def _final_body(a0_ref, g0_ref, a1_ref, g1_ref, deg0_ref, deg1_ref,
                b_ref, out_ref):
    dinv = lax.rsqrt(deg0_ref[0] + deg1_ref[0] + 1.0)
    p0 = dinv * (a0_ref[0] + g0_ref[0]) + b_ref[:, :H]
    p1 = dinv * (a1_ref[0] + g1_ref[0]) + b_ref[:, H:]
    out_ref[...] = jnp.concatenate([p0, p1], axis=1)


def _final(acc3, g3, deg3, b2d):
    return pl.pallas_call(
        _final_body,
        grid=(NBM,),
        in_specs=[
            pl.BlockSpec((1, BM, H), lambda m: (0, m, 0)),
            pl.BlockSpec((1, BM, H), lambda m: (0, m, 0)),
            pl.BlockSpec((1, BM, H), lambda m: (1, m, 0)),
            pl.BlockSpec((1, BM, H), lambda m: (1, m, 0)),
            pl.BlockSpec((1, BM, 1), lambda m: (0, m, 0)),
            pl.BlockSpec((1, BM, 1), lambda m: (1, m, 0)),
            pl.BlockSpec((1, D), lambda m: (0, 0)),
        ],
        out_specs=pl.BlockSpec((BM, D), lambda m: (m, 0)),
        out_shape=jax.ShapeDtypeStruct((N, D), jnp.float32),
        compiler_params=pltpu.CompilerParams(
            dimension_semantics=("parallel",)),
    )(acc3, g3, acc3, g3, deg3, deg3, b2d)


def kernel(x, edge_index, W1, b1, W2, b2, W3, b3):
    src = edge_index[0]
    dst = edge_index[1]
    zer = jnp.zeros((RPS, H), jnp.float32)
    src2 = jnp.concatenate([src, src + NP])

    deg3 = _deg(dst).reshape(NC, NP, 1)

    g3 = _mm1(x, W1, deg3)
    acc3 = _prop(g3.reshape(NC * NP, H), src2, dst, zer).reshape(NC, NP, H)
    g3 = _mmf(acc3, g3, b1.reshape(1, D), W2, deg3)
    acc3 = _prop(g3.reshape(NC * NP, H), src2, dst, zer).reshape(NC, NP, H)
    g3 = _mmf(acc3, g3, b2.reshape(1, D), W3, deg3)
    acc3 = _prop(g3.reshape(NC * NP, H), src2, dst, zer).reshape(NC, NP, H)

    return _final(acc3, g3, deg3, b3.reshape(1, D))
